# baseline (speedup 1.0000x reference)
import functools

import jax
import jax.numpy as jnp
from jax import lax
from jax.experimental import pallas as pl
from jax.experimental.pallas import tpu as pltpu
from jax.experimental.pallas import tpu_sc as plsc

N = 10000
E = 320000
C = 128
DE = 16
G = 64
L = 3

NB_N = 10
BN_ROWS = N // NB_N
BE_ROWS = 1000
NB_E = E // BE_ROWS

NW = 32
EPW = E // NW
K = 80
NCH = EPW // K
RPT = 624
RTAIL = N - 16 * RPT


def _embed_body(x_ref, w0_ref, w1_ref, o_ref):
    h = jnp.maximum(x_ref[...] @ w0_ref[...], 0.0)
    o_ref[...] = h @ w1_ref[...]


def _embed(x, w0, w1):
    return pl.pallas_call(
        _embed_body,
        grid=(NB_N,),
        in_specs=[pl.BlockSpec((BN_ROWS, C), lambda i: (i, 0)),
                  pl.BlockSpec((C, C), lambda i: (0, 0)),
                  pl.BlockSpec((C, C), lambda i: (0, 0))],
        out_specs=pl.BlockSpec((BN_ROWS, C), lambda i: (i, 0)),
        out_shape=jax.ShapeDtypeStruct((N, C), jnp.float32),
    )(x, w0, w1)


def _nt_body(h_ref, wd_ref, ws_ref, td_ref, ts_ref):
    h = h_ref[...]
    td_ref[...] = h @ wd_ref[...]
    ts_ref[...] = h @ ws_ref[...]


def _node_transform(h, wd, ws):
    return pl.pallas_call(
        _nt_body,
        grid=(NB_N,),
        in_specs=[pl.BlockSpec((BN_ROWS, C), lambda i: (i, 0)),
                  pl.BlockSpec((C, 2 * C), lambda i: (0, 0)),
                  pl.BlockSpec((C, 2 * C), lambda i: (0, 0))],
        out_specs=(pl.BlockSpec((BN_ROWS, 2 * C), lambda i: (i, 0)),
                   pl.BlockSpec((BN_ROWS, 2 * C), lambda i: (i, 0))),
        out_shape=(jax.ShapeDtypeStruct((N, 2 * C), jnp.float32),
                   jax.ShapeDtypeStruct((N, 2 * C), jnp.float32)),
    )(h, wd, ws)


def _ew_body(gd_ref, gs_ref, ea_ref, wfe_ref, wse_ref, bf_ref, bs_ref, o_ref):
    gd = gd_ref[...]
    gs = gs_ref[...]
    ea = ea_ref[...]
    f = gd[:, :C] + gs[:, :C] + ea @ wfe_ref[...] + bf_ref[...]
    s = gd[:, C:] + gs[:, C:] + ea @ wse_ref[...] + bs_ref[...]
    sig = 1.0 / (1.0 + jnp.exp(-f))
    sp = jnp.maximum(s, 0.0) + jnp.log(1.0 + jnp.exp(-jnp.abs(s)))
    o_ref[...] = sig * sp


def _edge_msg(gd, gs, ea, wfe, wse, bf, bs):
    ne = gd.shape[0]
    be = 640
    return pl.pallas_call(
        _ew_body,
        grid=(ne // be,),
        in_specs=[pl.BlockSpec((be, 2 * C), lambda i: (i, 0)),
                  pl.BlockSpec((be, 2 * C), lambda i: (i, 0)),
                  pl.BlockSpec((be, DE), lambda i: (i, 0)),
                  pl.BlockSpec((DE, C), lambda i: (0, 0)),
                  pl.BlockSpec((DE, C), lambda i: (0, 0)),
                  pl.BlockSpec((1, C), lambda i: (0, 0)),
                  pl.BlockSpec((1, C), lambda i: (0, 0))],
        out_specs=pl.BlockSpec((be, C), lambda i: (i, 0)),
        out_shape=jax.ShapeDtypeStruct((ne, C), jnp.float32),
    )(gd, gs, ea, wfe, wse, bf, bs)


def _agg_body(p0_ref, p1_ref, p2_ref, p3_ref, agg_ref, st_ref):
    a = (p0_ref[...] + p1_ref[...]) + (p2_ref[...] + p3_ref[...])
    agg_ref[...] = a

    @pl.when(pl.program_id(0) == 0)
    def _():
        st_ref[...] = jnp.zeros_like(st_ref)

    s = jnp.sum(a, axis=0, keepdims=True)
    q = jnp.sum(a * a, axis=0, keepdims=True)
    st_ref[...] += jnp.concatenate([s, q, jnp.zeros((6, C), jnp.float32)], axis=0)


def _agg_stats(parts_a, parts_b):
    return pl.pallas_call(
        _agg_body,
        grid=(NB_N,),
        in_specs=[pl.BlockSpec((BN_ROWS, C), lambda i: (i, 0)),
                  pl.BlockSpec((BN_ROWS, C), lambda i: (i + NB_N, 0)),
                  pl.BlockSpec((BN_ROWS, C), lambda i: (i, 0)),
                  pl.BlockSpec((BN_ROWS, C), lambda i: (i + NB_N, 0))],
        out_specs=(pl.BlockSpec((BN_ROWS, C), lambda i: (i, 0)),
                   pl.BlockSpec((8, C), lambda i: (0, 0))),
        out_shape=(jax.ShapeDtypeStruct((N, C), jnp.float32),
                   jax.ShapeDtypeStruct((8, C), jnp.float32)),
    )(parts_a, parts_a, parts_b, parts_b)


def _bn_body(agg_ref, h_ref, st_ref, gam_ref, bet_ref, b_ref, hn_ref, gf_ref):
    st = st_ref[...]
    mean = st[0:1, :] * (1.0 / N)
    var = st[1:2, :] * (1.0 / N) - mean * mean
    a = agg_ref[...]
    o = (a - mean) / jnp.sqrt(var + 1e-5) * gam_ref[...] + bet_ref[...] + h_ref[...]
    hn = jnp.maximum(o, 0.0)
    hn_ref[...] = hn

    @pl.when(pl.program_id(0) == 0)
    def _():
        gf_ref[...] = jnp.full((G, C), -jnp.inf, jnp.float32)

    b = b_ref[...]
    ms = [jnp.max(jnp.where(b == g, hn, -jnp.inf), axis=0, keepdims=True)
          for g in range(G)]
    gf_ref[...] = jnp.maximum(gf_ref[...], jnp.concatenate(ms, axis=0))


def _bn_relu_segmax(agg, h, st, gamma, beta, batch_f):
    return pl.pallas_call(
        _bn_body,
        grid=(NB_N,),
        in_specs=[pl.BlockSpec((BN_ROWS, C), lambda i: (i, 0)),
                  pl.BlockSpec((BN_ROWS, C), lambda i: (i, 0)),
                  pl.BlockSpec((8, C), lambda i: (0, 0)),
                  pl.BlockSpec((1, C), lambda i: (0, 0)),
                  pl.BlockSpec((1, C), lambda i: (0, 0)),
                  pl.BlockSpec((BN_ROWS, 1), lambda i: (i, 0))],
        out_specs=(pl.BlockSpec((BN_ROWS, C), lambda i: (i, 0)),
                   pl.BlockSpec((G, C), lambda i: (0, 0))),
        out_shape=(jax.ShapeDtypeStruct((N, C), jnp.float32),
                   jax.ShapeDtypeStruct((G, C), jnp.float32)),
    )(agg, h, st, gamma, beta, batch_f)


def _head_body(g0_ref, g1_ref, g2_ref, w1_ref, b1_ref, gam_ref, bet_ref,
               w2_ref, b2_ref, o_ref):
    gf = g0_ref[...] + g1_ref[...] + g2_ref[...]
    g = gf @ w1_ref[...] + b1_ref[...]
    m = jnp.mean(g, axis=0, keepdims=True)
    v = jnp.mean((g - m) ** 2, axis=0, keepdims=True)
    gr = jnp.maximum((g - m) / jnp.sqrt(v + 1e-5) * gam_ref[...] + bet_ref[...], 0.0)
    r = jnp.sum(gr * w2_ref[...], axis=1, keepdims=True) + b2_ref[0:1, 0:1]
    o_ref[...] = jnp.broadcast_to(r, (G, C))


def _head(g0, g1, g2, w1, b1, gam, bet, w2row, b2):
    full = lambda shape: pl.BlockSpec(shape, lambda: (0, 0))
    return pl.pallas_call(
        _head_body,
        in_specs=[full((G, C)), full((G, C)), full((G, C)), full((C, C)),
                  full((1, C)), full((1, C)), full((1, C)), full((1, C)),
                  full((8, C))],
        out_specs=full((G, C)),
        out_shape=jax.ShapeDtypeStruct((G, C), jnp.float32),
    )(g0, g1, g2, w1, b1, gam, bet, w2row, b2)


_MESH = plsc.VectorSubcoreMesh(core_axis_name="c", subcore_axis_name="s")

NCH_A = 63
NCH_B = NCH - NCH_A
EA = NCH_A * K * NW
EB = E - EA


def _make_gather(nch, off):
    ne = nch * K * NW
    epw = nch * K

    @functools.partial(
        pl.kernel,
        mesh=_MESH,
        out_type=(jax.ShapeDtypeStruct((ne, 2 * C), jnp.float32),
                  jax.ShapeDtypeStruct((ne, 2 * C), jnp.float32)),
        scratch_types=[
            pltpu.VMEM((epw,), jnp.int32),
            pltpu.VMEM((epw,), jnp.int32),
            pltpu.VMEM((K, 2 * C), jnp.float32),
            pltpu.VMEM((K, 2 * C), jnp.float32),
            pltpu.VMEM((K, 2 * C), jnp.float32),
            pltpu.VMEM((K, 2 * C), jnp.float32),
            pltpu.SemaphoreType.DMA,
            pltpu.SemaphoreType.DMA,
        ],
    )
    def gather_k(td, ts, dst, src, gd, gs, idx_d, idx_s, rd0, rs0, rd1, rs1,
                 g0, g1):
        cid = lax.axis_index("c")
        sid = lax.axis_index("s")
        wid = sid * 2 + cid
        wbase = wid * epw
        pltpu.sync_copy(dst.at[pl.ds(off + wbase, epw)], idx_d)
        pltpu.sync_copy(src.at[pl.ds(off + wbase, epw)], idx_s)

        def issue(c, rd, rs, sem):
            pltpu.async_copy(td.at[idx_d.at[pl.ds(c * K, K)]], rd, sem)
            pltpu.async_copy(ts.at[idx_s.at[pl.ds(c * K, K)]], rs, sem)

        def finish(c, rd, rs, sem):
            pltpu.make_async_copy(td.at[idx_d.at[pl.ds(0, K)]], rd, sem).wait()
            pltpu.make_async_copy(ts.at[idx_s.at[pl.ds(0, K)]], rs, sem).wait()
            pltpu.sync_copy(rd, gd.at[pl.ds(wbase + c * K, K)])
            pltpu.sync_copy(rs, gs.at[pl.ds(wbase + c * K, K)])

        issue(0, rd0, rs0, g0)
        issue(1, rd1, rs1, g1)

        def body(j, carry):
            c0 = 2 * j
            finish(c0, rd0, rs0, g0)

            @pl.when(c0 + 2 < nch)
            def _():
                issue(c0 + 2, rd0, rs0, g0)

            @pl.when(c0 + 1 < nch)
            def _():
                finish(c0 + 1, rd1, rs1, g1)

                @pl.when(c0 + 3 < nch)
                def _():
                    issue(c0 + 3, rd1, rs1, g1)

            return carry

        lax.fori_loop(0, (nch + 1) // 2, body, 0)

    return gather_k


def _make_scatter(nch):
    ne = nch * K * NW
    epw = nch * K

    @functools.partial(
        pl.kernel,
        mesh=_MESH,
        out_type=jax.ShapeDtypeStruct((2 * N, C), jnp.float32),
        scratch_types=[
            pltpu.VMEM((nch, K), jnp.int32),
            pltpu.VMEM((K, C), jnp.float32),
            pltpu.VMEM((K, C), jnp.float32),
            pltpu.VMEM_SHARED((N, C), jnp.float32),
            pltpu.SemaphoreType.DMA,
            pltpu.SemaphoreType.DMA,
        ],
    )
    def scatter_k(msg, dst_r, zeros, out, idx_v, m0, m1, acc, s0, s1):
        cid = lax.axis_index("c")
        sid = lax.axis_index("s")
        pltpu.sync_copy(dst_r.at[cid, sid], idx_v)
        pltpu.sync_copy(zeros.at[pl.ds(sid * RPT, RPT)],
                        acc.at[pl.ds(sid * RPT, RPT)])

        @pl.when(sid == 0)
        def _():
            pltpu.sync_copy(zeros.at[pl.ds(16 * RPT, RTAIL)],
                            acc.at[pl.ds(16 * RPT, RTAIL)])

        plsc.subcore_barrier()
        ebase = cid * (ne // 2) + sid * epw

        def load(c, m, sem):
            pltpu.async_copy(msg.at[pl.ds(ebase + c * K, K)], m, sem)

        def flush(c, m, sem):
            pltpu.make_async_copy(msg.at[pl.ds(ebase, K)], m, sem).wait()
            pltpu.sync_copy(m, acc.at[idx_v.at[c]], add=True)

        load(0, m0, s0)
        load(1, m1, s1)

        def body(j, carry):
            c0 = 2 * j
            flush(c0, m0, s0)

            @pl.when(c0 + 2 < nch)
            def _():
                load(c0 + 2, m0, s0)

            @pl.when(c0 + 1 < nch)
            def _():
                flush(c0 + 1, m1, s1)

                @pl.when(c0 + 3 < nch)
                def _():
                    load(c0 + 3, m1, s1)

            return carry

        lax.fori_loop(0, (nch + 1) // 2, body, 0)
        plsc.subcore_barrier()
        pltpu.sync_copy(acc.at[pl.ds(sid * RPT, RPT)],
                        out.at[pl.ds(cid * N + sid * RPT, RPT)])

        @pl.when(sid == 0)
        def _():
            pltpu.sync_copy(acc.at[pl.ds(16 * RPT, RTAIL)],
                            out.at[pl.ds(cid * N + 16 * RPT, RTAIL)])

    return scatter_k


_GATHER_A = _make_gather(NCH_A, 0)
_GATHER_B = _make_gather(NCH_B, EA)
_SCATTER_A = _make_scatter(NCH_A)
_SCATTER_B = _make_scatter(NCH_B)


def kernel(x, edge_index, edge_attr, batch, params):
    src = edge_index[0].astype(jnp.int32)
    dst = edge_index[1].astype(jnp.int32)
    dst_ra = dst[:EA].reshape(2, 16, NCH_A, K)
    dst_rb = dst[EA:].reshape(2, 16, NCH_B, K)
    ea_a = edge_attr[:EA]
    ea_b = edge_attr[EA:]
    batch_f = batch.astype(jnp.float32).reshape(N, 1)
    zeros = jnp.zeros((N, C), jnp.float32)

    h = _embed(x, params['emb_W0'], params['emb_W1'])
    gfs = []
    for l in range(L):
        wf = params[f'conv{l}_Wf']
        ws = params[f'conv{l}_Ws']
        wd = jnp.concatenate([wf[:C], ws[:C]], axis=1)
        wsrc = jnp.concatenate([wf[C:2 * C], ws[C:2 * C]], axis=1)
        bf = params[f'conv{l}_bf'].reshape(1, C)
        bs = params[f'conv{l}_bs'].reshape(1, C)
        td, ts = _node_transform(h, wd, wsrc)
        gd_a, gs_a = _GATHER_A(td, ts, dst, src)
        gd_b, gs_b = _GATHER_B(td, ts, dst, src)
        msg_a = _edge_msg(gd_a, gs_a, ea_a, wf[2 * C:], ws[2 * C:], bf, bs)
        msg_b = _edge_msg(gd_b, gs_b, ea_b, wf[2 * C:], ws[2 * C:], bf, bs)
        parts_a = _SCATTER_A(msg_a, dst_ra, zeros)
        parts_b = _SCATTER_B(msg_b, dst_rb, zeros)
        agg, st = _agg_stats(parts_a, parts_b)
        h, gf = _bn_relu_segmax(agg, h, st,
                                params[f'conv{l}_gamma'].reshape(1, C),
                                params[f'conv{l}_beta'].reshape(1, C),
                                batch_f)
        gfs.append(gf)

    out = _head(gfs[0], gfs[1], gfs[2], params['lin1_W'],
                params['lin1_b'].reshape(1, C),
                params['bn_gamma'].reshape(1, C),
                params['bn_beta'].reshape(1, C),
                params['lin2_W'].reshape(1, C),
                jnp.broadcast_to(params['lin2_b'].reshape(1, 1), (8, C)))
    return out[:, 0]


# speedup vs baseline: 1.1129x; 1.1129x over previous
import functools

import jax
import jax.numpy as jnp
from jax import lax
from jax.experimental import pallas as pl
from jax.experimental.pallas import tpu as pltpu
from jax.experimental.pallas import tpu_sc as plsc

N = 10000
E = 320000
C = 128
DE = 16
G = 64
L = 3

NB_N = 10
BN_ROWS = N // NB_N
BE_ROWS = 1000
NB_E = E // BE_ROWS

NW = 32
EPW = E // NW
K = 80
NCH = EPW // K
RPT = 624
RTAIL = N - 16 * RPT


def _embed_body(x_ref, w0_ref, w1_ref, o_ref):
    h = jnp.maximum(x_ref[...] @ w0_ref[...], 0.0)
    o_ref[...] = h @ w1_ref[...]


def _embed(x, w0, w1):
    return pl.pallas_call(
        _embed_body,
        grid=(NB_N,),
        in_specs=[pl.BlockSpec((BN_ROWS, C), lambda i: (i, 0)),
                  pl.BlockSpec((C, C), lambda i: (0, 0)),
                  pl.BlockSpec((C, C), lambda i: (0, 0))],
        out_specs=pl.BlockSpec((BN_ROWS, C), lambda i: (i, 0)),
        out_shape=jax.ShapeDtypeStruct((N, C), jnp.float32),
    )(x, w0, w1)


def _nt_body(h_ref, wd_ref, ws_ref, td_ref, ts_ref):
    h = h_ref[...]
    td_ref[...] = h @ wd_ref[...]
    ts_ref[...] = h @ ws_ref[...]


def _node_transform(h, wd, ws):
    return pl.pallas_call(
        _nt_body,
        grid=(NB_N,),
        in_specs=[pl.BlockSpec((BN_ROWS, C), lambda i: (i, 0)),
                  pl.BlockSpec((C, 2 * C), lambda i: (0, 0)),
                  pl.BlockSpec((C, 2 * C), lambda i: (0, 0))],
        out_specs=(pl.BlockSpec((BN_ROWS, 2 * C), lambda i: (i, 0)),
                   pl.BlockSpec((BN_ROWS, 2 * C), lambda i: (i, 0))),
        out_shape=(jax.ShapeDtypeStruct((N, 2 * C), jnp.float32),
                   jax.ShapeDtypeStruct((N, 2 * C), jnp.float32)),
    )(h, wd, ws)


def _ew_body(gd_ref, gs_ref, ea_ref, wfe_ref, wse_ref, bf_ref, bs_ref, o_ref):
    gd = gd_ref[...]
    gs = gs_ref[...]
    ea = ea_ref[...]
    f = gd[:, :C] + gs[:, :C] + ea @ wfe_ref[...] + bf_ref[...]
    s = gd[:, C:] + gs[:, C:] + ea @ wse_ref[...] + bs_ref[...]
    sig = 1.0 / (1.0 + jnp.exp(-f))
    sp = jnp.maximum(s, 0.0) + jnp.log(1.0 + jnp.exp(-jnp.abs(s)))
    o_ref[...] = sig * sp


def _edge_msg(gd, gs, ea, wfe, wse, bf, bs):
    ne = gd.shape[0]
    be = 2000
    return pl.pallas_call(
        _ew_body,
        grid=(ne // be,),
        in_specs=[pl.BlockSpec((be, 2 * C), lambda i: (i, 0)),
                  pl.BlockSpec((be, 2 * C), lambda i: (i, 0)),
                  pl.BlockSpec((be, DE), lambda i: (i, 0)),
                  pl.BlockSpec((DE, C), lambda i: (0, 0)),
                  pl.BlockSpec((DE, C), lambda i: (0, 0)),
                  pl.BlockSpec((1, C), lambda i: (0, 0)),
                  pl.BlockSpec((1, C), lambda i: (0, 0))],
        out_specs=pl.BlockSpec((be, C), lambda i: (i, 0)),
        out_shape=jax.ShapeDtypeStruct((ne, C), jnp.float32),
    )(gd, gs, ea, wfe, wse, bf, bs)


def _agg_body(p0_ref, p1_ref, agg_ref, st_ref):
    a = p0_ref[...] + p1_ref[...]
    agg_ref[...] = a

    @pl.when(pl.program_id(0) == 0)
    def _():
        st_ref[...] = jnp.zeros_like(st_ref)

    s = jnp.sum(a, axis=0, keepdims=True)
    q = jnp.sum(a * a, axis=0, keepdims=True)
    st_ref[...] += jnp.concatenate([s, q, jnp.zeros((6, C), jnp.float32)], axis=0)


def _agg_stats(parts):
    return pl.pallas_call(
        _agg_body,
        grid=(NB_N,),
        in_specs=[pl.BlockSpec((BN_ROWS, C), lambda i: (i, 0)),
                  pl.BlockSpec((BN_ROWS, C), lambda i: (i + NB_N, 0))],
        out_specs=(pl.BlockSpec((BN_ROWS, C), lambda i: (i, 0)),
                   pl.BlockSpec((8, C), lambda i: (0, 0))),
        out_shape=(jax.ShapeDtypeStruct((N, C), jnp.float32),
                   jax.ShapeDtypeStruct((8, C), jnp.float32)),
    )(parts, parts)


def _bn_body(agg_ref, h_ref, st_ref, gam_ref, bet_ref, b_ref, hn_ref, gf_ref):
    st = st_ref[...]
    mean = st[0:1, :] * (1.0 / N)
    var = st[1:2, :] * (1.0 / N) - mean * mean
    a = agg_ref[...]
    o = (a - mean) / jnp.sqrt(var + 1e-5) * gam_ref[...] + bet_ref[...] + h_ref[...]
    hn = jnp.maximum(o, 0.0)
    hn_ref[...] = hn

    @pl.when(pl.program_id(0) == 0)
    def _():
        gf_ref[...] = jnp.full((G, C), -jnp.inf, jnp.float32)

    b = b_ref[...]
    ms = [jnp.max(jnp.where(b == g, hn, -jnp.inf), axis=0, keepdims=True)
          for g in range(G)]
    gf_ref[...] = jnp.maximum(gf_ref[...], jnp.concatenate(ms, axis=0))


def _bn_relu_segmax(agg, h, st, gamma, beta, batch_f):
    return pl.pallas_call(
        _bn_body,
        grid=(NB_N,),
        in_specs=[pl.BlockSpec((BN_ROWS, C), lambda i: (i, 0)),
                  pl.BlockSpec((BN_ROWS, C), lambda i: (i, 0)),
                  pl.BlockSpec((8, C), lambda i: (0, 0)),
                  pl.BlockSpec((1, C), lambda i: (0, 0)),
                  pl.BlockSpec((1, C), lambda i: (0, 0)),
                  pl.BlockSpec((BN_ROWS, 1), lambda i: (i, 0))],
        out_specs=(pl.BlockSpec((BN_ROWS, C), lambda i: (i, 0)),
                   pl.BlockSpec((G, C), lambda i: (0, 0))),
        out_shape=(jax.ShapeDtypeStruct((N, C), jnp.float32),
                   jax.ShapeDtypeStruct((G, C), jnp.float32)),
    )(agg, h, st, gamma, beta, batch_f)


def _head_body(g0_ref, g1_ref, g2_ref, w1_ref, b1_ref, gam_ref, bet_ref,
               w2_ref, b2_ref, o_ref):
    gf = g0_ref[...] + g1_ref[...] + g2_ref[...]
    g = gf @ w1_ref[...] + b1_ref[...]
    m = jnp.mean(g, axis=0, keepdims=True)
    v = jnp.mean((g - m) ** 2, axis=0, keepdims=True)
    gr = jnp.maximum((g - m) / jnp.sqrt(v + 1e-5) * gam_ref[...] + bet_ref[...], 0.0)
    r = jnp.sum(gr * w2_ref[...], axis=1, keepdims=True) + b2_ref[0:1, 0:1]
    o_ref[...] = jnp.broadcast_to(r, (G, C))


def _head(g0, g1, g2, w1, b1, gam, bet, w2row, b2):
    full = lambda shape: pl.BlockSpec(shape, lambda: (0, 0))
    return pl.pallas_call(
        _head_body,
        in_specs=[full((G, C)), full((G, C)), full((G, C)), full((C, C)),
                  full((1, C)), full((1, C)), full((1, C)), full((1, C)),
                  full((8, C))],
        out_specs=full((G, C)),
        out_shape=jax.ShapeDtypeStruct((G, C), jnp.float32),
    )(g0, g1, g2, w1, b1, gam, bet, w2row, b2)


_MESH = plsc.VectorSubcoreMesh(core_axis_name="c", subcore_axis_name="s")

NCH_A = 63
NCH_B = NCH - NCH_A
EA = NCH_A * K * NW
EB = E - EA


def _make_gather(nch, off):
    ne = nch * K * NW
    epw = nch * K

    @functools.partial(
        pl.kernel,
        mesh=_MESH,
        out_type=(jax.ShapeDtypeStruct((ne, 2 * C), jnp.float32),
                  jax.ShapeDtypeStruct((ne, 2 * C), jnp.float32)),
        scratch_types=[
            pltpu.VMEM((epw,), jnp.int32),
            pltpu.VMEM((epw,), jnp.int32),
            pltpu.VMEM((K, 2 * C), jnp.float32),
            pltpu.VMEM((K, 2 * C), jnp.float32),
            pltpu.VMEM((K, 2 * C), jnp.float32),
            pltpu.VMEM((K, 2 * C), jnp.float32),
            pltpu.SemaphoreType.DMA,
            pltpu.SemaphoreType.DMA,
        ],
    )
    def gather_k(td, ts, dst, src, gd, gs, idx_d, idx_s, rd0, rs0, rd1, rs1,
                 g0, g1):
        cid = lax.axis_index("c")
        sid = lax.axis_index("s")
        wid = sid * 2 + cid
        wbase = wid * epw
        pltpu.sync_copy(dst.at[pl.ds(off + wbase, epw)], idx_d)
        pltpu.sync_copy(src.at[pl.ds(off + wbase, epw)], idx_s)

        def issue(c, rd, rs, sem):
            pltpu.async_copy(td.at[idx_d.at[pl.ds(c * K, K)]], rd, sem)
            pltpu.async_copy(ts.at[idx_s.at[pl.ds(c * K, K)]], rs, sem)

        def finish(c, rd, rs, sem):
            pltpu.make_async_copy(td.at[idx_d.at[pl.ds(0, K)]], rd, sem).wait()
            pltpu.make_async_copy(ts.at[idx_s.at[pl.ds(0, K)]], rs, sem).wait()
            pltpu.sync_copy(rd, gd.at[pl.ds(wbase + c * K, K)])
            pltpu.sync_copy(rs, gs.at[pl.ds(wbase + c * K, K)])

        issue(0, rd0, rs0, g0)
        issue(1, rd1, rs1, g1)

        def body(j, carry):
            c0 = 2 * j
            finish(c0, rd0, rs0, g0)

            @pl.when(c0 + 2 < nch)
            def _():
                issue(c0 + 2, rd0, rs0, g0)

            @pl.when(c0 + 1 < nch)
            def _():
                finish(c0 + 1, rd1, rs1, g1)

                @pl.when(c0 + 3 < nch)
                def _():
                    issue(c0 + 3, rd1, rs1, g1)

            return carry

        lax.fori_loop(0, (nch + 1) // 2, body, 0)

    return gather_k


def _make_scatter(nch):
    ne = nch * K * NW
    epw = nch * K

    @functools.partial(
        pl.kernel,
        mesh=_MESH,
        out_type=jax.ShapeDtypeStruct((2 * N, C), jnp.float32),
        scratch_types=[
            pltpu.VMEM((nch, K), jnp.int32),
            pltpu.VMEM((K, C), jnp.float32),
            pltpu.VMEM((K, C), jnp.float32),
            pltpu.VMEM_SHARED((N, C), jnp.float32),
            pltpu.SemaphoreType.DMA,
            pltpu.SemaphoreType.DMA,
        ],
    )
    def scatter_k(msg, dst_r, zeros, out, idx_v, m0, m1, acc, s0, s1):
        cid = lax.axis_index("c")
        sid = lax.axis_index("s")
        pltpu.sync_copy(dst_r.at[cid, sid], idx_v)
        pltpu.sync_copy(zeros.at[pl.ds(sid * RPT, RPT)],
                        acc.at[pl.ds(sid * RPT, RPT)])

        @pl.when(sid == 0)
        def _():
            pltpu.sync_copy(zeros.at[pl.ds(16 * RPT, RTAIL)],
                            acc.at[pl.ds(16 * RPT, RTAIL)])

        plsc.subcore_barrier()
        ebase = cid * (ne // 2) + sid * epw

        def load(c, m, sem):
            pltpu.async_copy(msg.at[pl.ds(ebase + c * K, K)], m, sem)

        def flush(c, m, sem):
            pltpu.make_async_copy(msg.at[pl.ds(ebase, K)], m, sem).wait()
            pltpu.sync_copy(m, acc.at[idx_v.at[c]], add=True)

        load(0, m0, s0)
        load(1, m1, s1)

        def body(j, carry):
            c0 = 2 * j
            flush(c0, m0, s0)

            @pl.when(c0 + 2 < nch)
            def _():
                load(c0 + 2, m0, s0)

            @pl.when(c0 + 1 < nch)
            def _():
                flush(c0 + 1, m1, s1)

                @pl.when(c0 + 3 < nch)
                def _():
                    load(c0 + 3, m1, s1)

            return carry

        lax.fori_loop(0, (nch + 1) // 2, body, 0)
        plsc.subcore_barrier()
        pltpu.sync_copy(acc.at[pl.ds(sid * RPT, RPT)],
                        out.at[pl.ds(cid * N + sid * RPT, RPT)])

        @pl.when(sid == 0)
        def _():
            pltpu.sync_copy(acc.at[pl.ds(16 * RPT, RTAIL)],
                            out.at[pl.ds(cid * N + 16 * RPT, RTAIL)])

    return scatter_k


_GATHER_F = _make_gather(NCH, 0)
_SCATTER_F = _make_scatter(NCH)


def kernel(x, edge_index, edge_attr, batch, params):
    src = edge_index[0].astype(jnp.int32)
    dst = edge_index[1].astype(jnp.int32)
    dst_r = dst.reshape(2, 16, NCH, K)
    batch_f = batch.astype(jnp.float32).reshape(N, 1)
    zeros = jnp.zeros((N, C), jnp.float32)

    h = _embed(x, params['emb_W0'], params['emb_W1'])
    gfs = []
    for l in range(L):
        wf = params[f'conv{l}_Wf']
        ws = params[f'conv{l}_Ws']
        wd = jnp.concatenate([wf[:C], ws[:C]], axis=1)
        wsrc = jnp.concatenate([wf[C:2 * C], ws[C:2 * C]], axis=1)
        bf = params[f'conv{l}_bf'].reshape(1, C)
        bs = params[f'conv{l}_bs'].reshape(1, C)
        td, ts = _node_transform(h, wd, wsrc)
        gd, gs = _GATHER_F(td, ts, dst, src)
        msg = _edge_msg(gd, gs, edge_attr, wf[2 * C:], ws[2 * C:], bf, bs)
        parts = _SCATTER_F(msg, dst_r, zeros)
        agg, st = _agg_stats(parts)
        h, gf = _bn_relu_segmax(agg, h, st,
                                params[f'conv{l}_gamma'].reshape(1, C),
                                params[f'conv{l}_beta'].reshape(1, C),
                                batch_f)
        gfs.append(gf)

    out = _head(gfs[0], gfs[1], gfs[2], params['lin1_W'],
                params['lin1_b'].reshape(1, C),
                params['bn_gamma'].reshape(1, C),
                params['bn_beta'].reshape(1, C),
                params['lin2_W'].reshape(1, C),
                jnp.broadcast_to(params['lin2_b'].reshape(1, 1), (8, C)))
    return out[:, 0]


# speedup vs baseline: 1.1170x; 1.0037x over previous
import functools

import jax
import jax.numpy as jnp
from jax import lax
from jax.experimental import pallas as pl
from jax.experimental.pallas import tpu as pltpu
from jax.experimental.pallas import tpu_sc as plsc

N = 10000
E = 320000
C = 128
DE = 16
G = 64
L = 3

NB_N = 10
BN_ROWS = N // NB_N
BE_ROWS = 1000
NB_E = E // BE_ROWS

NW = 32
EPW = E // NW
K = 80
NCH = EPW // K
RPT = 624
RTAIL = N - 16 * RPT


def _embed_body(x_ref, w0_ref, w1_ref, wd_ref, wsrc_ref, h_ref, td_ref, ts_ref):
    t = jnp.maximum(x_ref[...] @ w0_ref[...], 0.0)
    h = t @ w1_ref[...]
    h_ref[...] = h
    td_ref[...] = h @ wd_ref[...]
    ts_ref[...] = h @ wsrc_ref[...]


def _embed(x, w0, w1, wd, wsrc):
    return pl.pallas_call(
        _embed_body,
        grid=(NB_N,),
        in_specs=[pl.BlockSpec((BN_ROWS, C), lambda i: (i, 0)),
                  pl.BlockSpec((C, C), lambda i: (0, 0)),
                  pl.BlockSpec((C, C), lambda i: (0, 0)),
                  pl.BlockSpec((C, 2 * C), lambda i: (0, 0)),
                  pl.BlockSpec((C, 2 * C), lambda i: (0, 0))],
        out_specs=(pl.BlockSpec((BN_ROWS, C), lambda i: (i, 0)),
                   pl.BlockSpec((BN_ROWS, 2 * C), lambda i: (i, 0)),
                   pl.BlockSpec((BN_ROWS, 2 * C), lambda i: (i, 0))),
        out_shape=(jax.ShapeDtypeStruct((N, C), jnp.float32),
                   jax.ShapeDtypeStruct((N, 2 * C), jnp.float32),
                   jax.ShapeDtypeStruct((N, 2 * C), jnp.float32)),
    )(x, w0, w1, wd, wsrc)


def _nt_body(h_ref, wd_ref, ws_ref, td_ref, ts_ref):
    h = h_ref[...]
    td_ref[...] = h @ wd_ref[...]
    ts_ref[...] = h @ ws_ref[...]


def _node_transform(h, wd, ws):
    return pl.pallas_call(
        _nt_body,
        grid=(NB_N,),
        in_specs=[pl.BlockSpec((BN_ROWS, C), lambda i: (i, 0)),
                  pl.BlockSpec((C, 2 * C), lambda i: (0, 0)),
                  pl.BlockSpec((C, 2 * C), lambda i: (0, 0))],
        out_specs=(pl.BlockSpec((BN_ROWS, 2 * C), lambda i: (i, 0)),
                   pl.BlockSpec((BN_ROWS, 2 * C), lambda i: (i, 0))),
        out_shape=(jax.ShapeDtypeStruct((N, 2 * C), jnp.float32),
                   jax.ShapeDtypeStruct((N, 2 * C), jnp.float32)),
    )(h, wd, ws)


def _ew_body(gd_ref, gs_ref, ea_ref, wfe_ref, wse_ref, bf_ref, bs_ref, o_ref):
    gd = gd_ref[...]
    gs = gs_ref[...]
    ea = ea_ref[...]
    f = gd[:, :C] + gs[:, :C] + ea @ wfe_ref[...] + bf_ref[...]
    s = gd[:, C:] + gs[:, C:] + ea @ wse_ref[...] + bs_ref[...]
    sig = 1.0 / (1.0 + jnp.exp(-f))
    sp = jnp.maximum(s, 0.0) + jnp.log(1.0 + jnp.exp(-jnp.abs(s)))
    o_ref[...] = sig * sp


def _edge_msg(gd, gs, ea, wfe, wse, bf, bs):
    ne = gd.shape[0]
    be = 2000
    return pl.pallas_call(
        _ew_body,
        grid=(ne // be,),
        in_specs=[pl.BlockSpec((be, 2 * C), lambda i: (i, 0)),
                  pl.BlockSpec((be, 2 * C), lambda i: (i, 0)),
                  pl.BlockSpec((be, DE), lambda i: (i, 0)),
                  pl.BlockSpec((DE, C), lambda i: (0, 0)),
                  pl.BlockSpec((DE, C), lambda i: (0, 0)),
                  pl.BlockSpec((1, C), lambda i: (0, 0)),
                  pl.BlockSpec((1, C), lambda i: (0, 0))],
        out_specs=pl.BlockSpec((be, C), lambda i: (i, 0)),
        out_shape=jax.ShapeDtypeStruct((ne, C), jnp.float32),
    )(gd, gs, ea, wfe, wse, bf, bs)


def _agg_body(p0_ref, p1_ref, agg_ref, st_ref):
    a = p0_ref[...] + p1_ref[...]
    agg_ref[...] = a

    @pl.when(pl.program_id(0) == 0)
    def _():
        st_ref[...] = jnp.zeros_like(st_ref)

    s = jnp.sum(a, axis=0, keepdims=True)
    q = jnp.sum(a * a, axis=0, keepdims=True)
    st_ref[...] += jnp.concatenate([s, q, jnp.zeros((6, C), jnp.float32)], axis=0)


def _agg_stats(parts):
    return pl.pallas_call(
        _agg_body,
        grid=(NB_N,),
        in_specs=[pl.BlockSpec((BN_ROWS, C), lambda i: (i, 0)),
                  pl.BlockSpec((BN_ROWS, C), lambda i: (i + NB_N, 0))],
        out_specs=(pl.BlockSpec((BN_ROWS, C), lambda i: (i, 0)),
                   pl.BlockSpec((8, C), lambda i: (0, 0))),
        out_shape=(jax.ShapeDtypeStruct((N, C), jnp.float32),
                   jax.ShapeDtypeStruct((8, C), jnp.float32)),
    )(parts, parts)


def _bn_nt_body(p0_ref, p1_ref, h_ref, gam_ref, bet_ref, b_ref, wd_ref,
                wsrc_ref, hn_ref, gf_ref, td_ref, ts_ref, st_scr):
    p = pl.program_id(0)
    i = pl.program_id(1)
    a = p0_ref[...] + p1_ref[...]

    @pl.when((p == 0) & (i == 0))
    def _():
        st_scr[...] = jnp.zeros_like(st_scr)

    @pl.when(p == 0)
    def _():
        s = jnp.sum(a, axis=0, keepdims=True)
        q = jnp.sum(a * a, axis=0, keepdims=True)
        st_scr[...] += jnp.concatenate(
            [s, q, jnp.zeros((6, C), jnp.float32)], axis=0)

    @pl.when(p == 1)
    def _():
        st = st_scr[...]
        mean = st[0:1, :] * (1.0 / N)
        var = st[1:2, :] * (1.0 / N) - mean * mean
        o = (a - mean) / jnp.sqrt(var + 1e-5) * gam_ref[...] + bet_ref[...] \
            + h_ref[...]
        hn = jnp.maximum(o, 0.0)
        hn_ref[...] = hn
        td_ref[...] = hn @ wd_ref[...]
        ts_ref[...] = hn @ wsrc_ref[...]

        @pl.when(i == 0)
        def _():
            gf_ref[...] = jnp.full((G, C), -jnp.inf, jnp.float32)

        b = b_ref[...]
        ms = [jnp.max(jnp.where(b == g, hn, -jnp.inf), axis=0, keepdims=True)
              for g in range(G)]
        gf_ref[...] = jnp.maximum(gf_ref[...], jnp.concatenate(ms, axis=0))


def _bn_nt(parts, h, gamma, beta, batch_f, wd, wsrc):
    return pl.pallas_call(
        _bn_nt_body,
        grid=(2, NB_N),
        in_specs=[pl.BlockSpec((BN_ROWS, C), lambda p, i: (i, 0)),
                  pl.BlockSpec((BN_ROWS, C), lambda p, i: (i + NB_N, 0)),
                  pl.BlockSpec((BN_ROWS, C), lambda p, i: (i, 0)),
                  pl.BlockSpec((1, C), lambda p, i: (0, 0)),
                  pl.BlockSpec((1, C), lambda p, i: (0, 0)),
                  pl.BlockSpec((BN_ROWS, 1), lambda p, i: (i, 0)),
                  pl.BlockSpec((C, 2 * C), lambda p, i: (0, 0)),
                  pl.BlockSpec((C, 2 * C), lambda p, i: (0, 0))],
        out_specs=(pl.BlockSpec((BN_ROWS, C), lambda p, i: (i, 0)),
                   pl.BlockSpec((G, C), lambda p, i: (0, 0)),
                   pl.BlockSpec((BN_ROWS, 2 * C), lambda p, i: (i, 0)),
                   pl.BlockSpec((BN_ROWS, 2 * C), lambda p, i: (i, 0))),
        out_shape=(jax.ShapeDtypeStruct((N, C), jnp.float32),
                   jax.ShapeDtypeStruct((G, C), jnp.float32),
                   jax.ShapeDtypeStruct((N, 2 * C), jnp.float32),
                   jax.ShapeDtypeStruct((N, 2 * C), jnp.float32)),
        scratch_shapes=[pltpu.VMEM((8, C), jnp.float32)],
    )(parts, parts, h, gamma, beta, batch_f, wd, wsrc)


def _bn_body(agg_ref, h_ref, st_ref, gam_ref, bet_ref, b_ref, hn_ref, gf_ref):
    st = st_ref[...]
    mean = st[0:1, :] * (1.0 / N)
    var = st[1:2, :] * (1.0 / N) - mean * mean
    a = agg_ref[...]
    o = (a - mean) / jnp.sqrt(var + 1e-5) * gam_ref[...] + bet_ref[...] + h_ref[...]
    hn = jnp.maximum(o, 0.0)
    hn_ref[...] = hn

    @pl.when(pl.program_id(0) == 0)
    def _():
        gf_ref[...] = jnp.full((G, C), -jnp.inf, jnp.float32)

    b = b_ref[...]
    ms = [jnp.max(jnp.where(b == g, hn, -jnp.inf), axis=0, keepdims=True)
          for g in range(G)]
    gf_ref[...] = jnp.maximum(gf_ref[...], jnp.concatenate(ms, axis=0))


def _bn_relu_segmax(agg, h, st, gamma, beta, batch_f):
    return pl.pallas_call(
        _bn_body,
        grid=(NB_N,),
        in_specs=[pl.BlockSpec((BN_ROWS, C), lambda i: (i, 0)),
                  pl.BlockSpec((BN_ROWS, C), lambda i: (i, 0)),
                  pl.BlockSpec((8, C), lambda i: (0, 0)),
                  pl.BlockSpec((1, C), lambda i: (0, 0)),
                  pl.BlockSpec((1, C), lambda i: (0, 0)),
                  pl.BlockSpec((BN_ROWS, 1), lambda i: (i, 0))],
        out_specs=(pl.BlockSpec((BN_ROWS, C), lambda i: (i, 0)),
                   pl.BlockSpec((G, C), lambda i: (0, 0))),
        out_shape=(jax.ShapeDtypeStruct((N, C), jnp.float32),
                   jax.ShapeDtypeStruct((G, C), jnp.float32)),
    )(agg, h, st, gamma, beta, batch_f)


def _head_body(g0_ref, g1_ref, g2_ref, w1_ref, b1_ref, gam_ref, bet_ref,
               w2_ref, b2_ref, o_ref):
    gf = g0_ref[...] + g1_ref[...] + g2_ref[...]
    g = gf @ w1_ref[...] + b1_ref[...]
    m = jnp.mean(g, axis=0, keepdims=True)
    v = jnp.mean((g - m) ** 2, axis=0, keepdims=True)
    gr = jnp.maximum((g - m) / jnp.sqrt(v + 1e-5) * gam_ref[...] + bet_ref[...], 0.0)
    r = jnp.sum(gr * w2_ref[...], axis=1, keepdims=True) + b2_ref[0:1, 0:1]
    o_ref[...] = jnp.broadcast_to(r, (G, C))


def _head(g0, g1, g2, w1, b1, gam, bet, w2row, b2):
    full = lambda shape: pl.BlockSpec(shape, lambda: (0, 0))
    return pl.pallas_call(
        _head_body,
        in_specs=[full((G, C)), full((G, C)), full((G, C)), full((C, C)),
                  full((1, C)), full((1, C)), full((1, C)), full((1, C)),
                  full((8, C))],
        out_specs=full((G, C)),
        out_shape=jax.ShapeDtypeStruct((G, C), jnp.float32),
    )(g0, g1, g2, w1, b1, gam, bet, w2row, b2)


_MESH = plsc.VectorSubcoreMesh(core_axis_name="c", subcore_axis_name="s")

NCH_A = 63
NCH_B = NCH - NCH_A
EA = NCH_A * K * NW
EB = E - EA


def _make_gather(nch, off):
    ne = nch * K * NW
    epw = nch * K

    @functools.partial(
        pl.kernel,
        mesh=_MESH,
        out_type=(jax.ShapeDtypeStruct((ne, 2 * C), jnp.float32),
                  jax.ShapeDtypeStruct((ne, 2 * C), jnp.float32)),
        scratch_types=[
            pltpu.VMEM((epw,), jnp.int32),
            pltpu.VMEM((epw,), jnp.int32),
            pltpu.VMEM((K, 2 * C), jnp.float32),
            pltpu.VMEM((K, 2 * C), jnp.float32),
            pltpu.VMEM((K, 2 * C), jnp.float32),
            pltpu.VMEM((K, 2 * C), jnp.float32),
            pltpu.SemaphoreType.DMA,
            pltpu.SemaphoreType.DMA,
        ],
    )
    def gather_k(td, ts, dst, src, gd, gs, idx_d, idx_s, rd0, rs0, rd1, rs1,
                 g0, g1):
        cid = lax.axis_index("c")
        sid = lax.axis_index("s")
        wid = sid * 2 + cid
        wbase = wid * epw
        pltpu.sync_copy(dst.at[pl.ds(off + wbase, epw)], idx_d)
        pltpu.sync_copy(src.at[pl.ds(off + wbase, epw)], idx_s)

        def issue(c, rd, rs, sem):
            pltpu.async_copy(td.at[idx_d.at[pl.ds(c * K, K)]], rd, sem)
            pltpu.async_copy(ts.at[idx_s.at[pl.ds(c * K, K)]], rs, sem)

        def finish(c, rd, rs, sem):
            pltpu.make_async_copy(td.at[idx_d.at[pl.ds(0, K)]], rd, sem).wait()
            pltpu.make_async_copy(ts.at[idx_s.at[pl.ds(0, K)]], rs, sem).wait()
            pltpu.sync_copy(rd, gd.at[pl.ds(wbase + c * K, K)])
            pltpu.sync_copy(rs, gs.at[pl.ds(wbase + c * K, K)])

        issue(0, rd0, rs0, g0)
        issue(1, rd1, rs1, g1)

        def body(j, carry):
            c0 = 2 * j
            finish(c0, rd0, rs0, g0)

            @pl.when(c0 + 2 < nch)
            def _():
                issue(c0 + 2, rd0, rs0, g0)

            @pl.when(c0 + 1 < nch)
            def _():
                finish(c0 + 1, rd1, rs1, g1)

                @pl.when(c0 + 3 < nch)
                def _():
                    issue(c0 + 3, rd1, rs1, g1)

            return carry

        lax.fori_loop(0, (nch + 1) // 2, body, 0)

    return gather_k


def _make_scatter(nch):
    ne = nch * K * NW
    epw = nch * K

    @functools.partial(
        pl.kernel,
        mesh=_MESH,
        out_type=jax.ShapeDtypeStruct((2 * N, C), jnp.float32),
        scratch_types=[
            pltpu.VMEM((nch, K), jnp.int32),
            pltpu.VMEM((K, C), jnp.float32),
            pltpu.VMEM((K, C), jnp.float32),
            pltpu.VMEM_SHARED((N, C), jnp.float32),
            pltpu.SemaphoreType.DMA,
            pltpu.SemaphoreType.DMA,
        ],
    )
    def scatter_k(msg, dst_r, zeros, out, idx_v, m0, m1, acc, s0, s1):
        cid = lax.axis_index("c")
        sid = lax.axis_index("s")
        pltpu.sync_copy(dst_r.at[cid, sid], idx_v)
        pltpu.sync_copy(zeros.at[pl.ds(sid * RPT, RPT)],
                        acc.at[pl.ds(sid * RPT, RPT)])

        @pl.when(sid == 0)
        def _():
            pltpu.sync_copy(zeros.at[pl.ds(16 * RPT, RTAIL)],
                            acc.at[pl.ds(16 * RPT, RTAIL)])

        plsc.subcore_barrier()
        ebase = cid * (ne // 2) + sid * epw

        def load(c, m, sem):
            pltpu.async_copy(msg.at[pl.ds(ebase + c * K, K)], m, sem)

        def flush(c, m, sem):
            pltpu.make_async_copy(msg.at[pl.ds(ebase, K)], m, sem).wait()
            pltpu.sync_copy(m, acc.at[idx_v.at[c]], add=True)

        load(0, m0, s0)
        load(1, m1, s1)

        def body(j, carry):
            c0 = 2 * j
            flush(c0, m0, s0)

            @pl.when(c0 + 2 < nch)
            def _():
                load(c0 + 2, m0, s0)

            @pl.when(c0 + 1 < nch)
            def _():
                flush(c0 + 1, m1, s1)

                @pl.when(c0 + 3 < nch)
                def _():
                    load(c0 + 3, m1, s1)

            return carry

        lax.fori_loop(0, (nch + 1) // 2, body, 0)
        plsc.subcore_barrier()
        pltpu.sync_copy(acc.at[pl.ds(sid * RPT, RPT)],
                        out.at[pl.ds(cid * N + sid * RPT, RPT)])

        @pl.when(sid == 0)
        def _():
            pltpu.sync_copy(acc.at[pl.ds(16 * RPT, RTAIL)],
                            out.at[pl.ds(cid * N + 16 * RPT, RTAIL)])

    return scatter_k


_GATHER_F = _make_gather(NCH, 0)
_SCATTER_F = _make_scatter(NCH)


def kernel(x, edge_index, edge_attr, batch, params):
    src = edge_index[0].astype(jnp.int32)
    dst = edge_index[1].astype(jnp.int32)
    dst_r = dst.reshape(2, 16, NCH, K)
    batch_f = batch.astype(jnp.float32).reshape(N, 1)
    zeros = jnp.zeros((N, C), jnp.float32)

    wds = []
    wsrcs = []
    for l in range(L):
        wf = params[f'conv{l}_Wf']
        ws = params[f'conv{l}_Ws']
        wds.append(jnp.concatenate([wf[:C], ws[:C]], axis=1))
        wsrcs.append(jnp.concatenate([wf[C:2 * C], ws[C:2 * C]], axis=1))

    h, td, ts = _embed(x, params['emb_W0'], params['emb_W1'], wds[0], wsrcs[0])
    gfs = []
    for l in range(L):
        wf = params[f'conv{l}_Wf']
        ws = params[f'conv{l}_Ws']
        bf = params[f'conv{l}_bf'].reshape(1, C)
        bs = params[f'conv{l}_bs'].reshape(1, C)
        gamma = params[f'conv{l}_gamma'].reshape(1, C)
        beta = params[f'conv{l}_beta'].reshape(1, C)
        gd, gs = _GATHER_F(td, ts, dst, src)
        msg = _edge_msg(gd, gs, edge_attr, wf[2 * C:], ws[2 * C:], bf, bs)
        parts = _SCATTER_F(msg, dst_r, zeros)
        if l < L - 1:
            h, gf, td, ts = _bn_nt(parts, h, gamma, beta, batch_f,
                                   wds[l + 1], wsrcs[l + 1])
        else:
            agg, st = _agg_stats(parts)
            h, gf = _bn_relu_segmax(agg, h, st, gamma, beta, batch_f)
        gfs.append(gf)

    out = _head(gfs[0], gfs[1], gfs[2], params['lin1_W'],
                params['lin1_b'].reshape(1, C),
                params['bn_gamma'].reshape(1, C),
                params['bn_beta'].reshape(1, C),
                params['lin2_W'].reshape(1, C),
                jnp.broadcast_to(params['lin2_b'].reshape(1, 1), (8, C)))
    return out[:, 0]


# speedup vs baseline: 1.1180x; 1.0009x over previous
import functools

import jax
import jax.numpy as jnp
from jax import lax
from jax.experimental import pallas as pl
from jax.experimental.pallas import tpu as pltpu
from jax.experimental.pallas import tpu_sc as plsc

N = 10000
E = 320000
C = 128
DE = 16
G = 64
L = 3

NB_N = 10
BN_ROWS = N // NB_N
BE_ROWS = 1000
NB_E = E // BE_ROWS

NW = 32
EPW = E // NW
K = 80
NCH = EPW // K
RPT = 624
RTAIL = N - 16 * RPT


def _embed_body(x_ref, w0_ref, w1_ref, wd_ref, wsrc_ref, h_ref, td_ref, ts_ref):
    t = jnp.maximum(x_ref[...] @ w0_ref[...], 0.0)
    h = t @ w1_ref[...]
    h_ref[...] = h
    td_ref[...] = h @ wd_ref[...]
    ts_ref[...] = h @ wsrc_ref[...]


def _embed(x, w0, w1, wd, wsrc):
    return pl.pallas_call(
        _embed_body,
        grid=(NB_N,),
        in_specs=[pl.BlockSpec((BN_ROWS, C), lambda i: (i, 0)),
                  pl.BlockSpec((C, C), lambda i: (0, 0)),
                  pl.BlockSpec((C, C), lambda i: (0, 0)),
                  pl.BlockSpec((C, 2 * C), lambda i: (0, 0)),
                  pl.BlockSpec((C, 2 * C), lambda i: (0, 0))],
        out_specs=(pl.BlockSpec((BN_ROWS, C), lambda i: (i, 0)),
                   pl.BlockSpec((BN_ROWS, 2 * C), lambda i: (i, 0)),
                   pl.BlockSpec((BN_ROWS, 2 * C), lambda i: (i, 0))),
        out_shape=(jax.ShapeDtypeStruct((N, C), jnp.float32),
                   jax.ShapeDtypeStruct((N, 2 * C), jnp.float32),
                   jax.ShapeDtypeStruct((N, 2 * C), jnp.float32)),
    )(x, w0, w1, wd, wsrc)


def _ew_body(gd_ref, gs_ref, ea_ref, wfe_ref, wse_ref, bf_ref, bs_ref, o_ref):
    gd = gd_ref[...]
    gs = gs_ref[...]
    ea = ea_ref[...]
    f = gd[:, :C] + gs[:, :C] + ea @ wfe_ref[...] + bf_ref[...]
    s = gd[:, C:] + gs[:, C:] + ea @ wse_ref[...] + bs_ref[...]
    sig = 1.0 / (1.0 + jnp.exp(-f))
    sp = jnp.maximum(s, 0.0) + jnp.log(1.0 + jnp.exp(-jnp.abs(s)))
    o_ref[...] = sig * sp


def _edge_msg(gd, gs, ea, wfe, wse, bf, bs):
    ne = gd.shape[0]
    be = 2000
    return pl.pallas_call(
        _ew_body,
        grid=(ne // be,),
        in_specs=[pl.BlockSpec((be, 2 * C), lambda i: (i, 0)),
                  pl.BlockSpec((be, 2 * C), lambda i: (i, 0)),
                  pl.BlockSpec((be, DE), lambda i: (i, 0)),
                  pl.BlockSpec((DE, C), lambda i: (0, 0)),
                  pl.BlockSpec((DE, C), lambda i: (0, 0)),
                  pl.BlockSpec((1, C), lambda i: (0, 0)),
                  pl.BlockSpec((1, C), lambda i: (0, 0))],
        out_specs=pl.BlockSpec((be, C), lambda i: (i, 0)),
        out_shape=jax.ShapeDtypeStruct((ne, C), jnp.float32),
    )(gd, gs, ea, wfe, wse, bf, bs)


def _agg_body(p0_ref, p1_ref, agg_ref, st_ref):
    a = p0_ref[...] + p1_ref[...]
    agg_ref[...] = a

    @pl.when(pl.program_id(0) == 0)
    def _():
        st_ref[...] = jnp.zeros_like(st_ref)

    s = jnp.sum(a, axis=0, keepdims=True)
    q = jnp.sum(a * a, axis=0, keepdims=True)
    st_ref[...] += jnp.concatenate([s, q, jnp.zeros((6, C), jnp.float32)], axis=0)


def _agg_stats(parts):
    return pl.pallas_call(
        _agg_body,
        grid=(NB_N,),
        in_specs=[pl.BlockSpec((BN_ROWS, C), lambda i: (i, 0)),
                  pl.BlockSpec((BN_ROWS, C), lambda i: (i + NB_N, 0))],
        out_specs=(pl.BlockSpec((BN_ROWS, C), lambda i: (i, 0)),
                   pl.BlockSpec((8, C), lambda i: (0, 0))),
        out_shape=(jax.ShapeDtypeStruct((N, C), jnp.float32),
                   jax.ShapeDtypeStruct((8, C), jnp.float32)),
    )(parts, parts)


def _bn_nt_body(p0_ref, p1_ref, h_ref, gam_ref, bet_ref, b_ref, wd_ref,
                wsrc_ref, hn_ref, gf_ref, td_ref, ts_ref, st_scr):
    p = pl.program_id(0)
    i = pl.program_id(1)
    a = p0_ref[...] + p1_ref[...]

    @pl.when((p == 0) & (i == 0))
    def _():
        st_scr[...] = jnp.zeros_like(st_scr)

    @pl.when(p == 0)
    def _():
        s = jnp.sum(a, axis=0, keepdims=True)
        q = jnp.sum(a * a, axis=0, keepdims=True)
        st_scr[...] += jnp.concatenate(
            [s, q, jnp.zeros((6, C), jnp.float32)], axis=0)

    @pl.when(p == 1)
    def _():
        st = st_scr[...]
        mean = st[0:1, :] * (1.0 / N)
        var = st[1:2, :] * (1.0 / N) - mean * mean
        o = (a - mean) / jnp.sqrt(var + 1e-5) * gam_ref[...] + bet_ref[...] \
            + h_ref[...]
        hn = jnp.maximum(o, 0.0)
        hn_ref[...] = hn
        td_ref[...] = hn @ wd_ref[...]
        ts_ref[...] = hn @ wsrc_ref[...]

        @pl.when(i == 0)
        def _():
            gf_ref[...] = jnp.full((G, C), -jnp.inf, jnp.float32)

        b = b_ref[...]
        ms = [jnp.max(jnp.where(b == g, hn, -jnp.inf), axis=0, keepdims=True)
              for g in range(G)]
        gf_ref[...] = jnp.maximum(gf_ref[...], jnp.concatenate(ms, axis=0))


def _bn_nt(parts, h, gamma, beta, batch_f, wd, wsrc):
    return pl.pallas_call(
        _bn_nt_body,
        grid=(2, NB_N),
        in_specs=[pl.BlockSpec((BN_ROWS, C), lambda p, i: (i, 0)),
                  pl.BlockSpec((BN_ROWS, C), lambda p, i: (i + NB_N, 0)),
                  pl.BlockSpec((BN_ROWS, C), lambda p, i: (i, 0)),
                  pl.BlockSpec((1, C), lambda p, i: (0, 0)),
                  pl.BlockSpec((1, C), lambda p, i: (0, 0)),
                  pl.BlockSpec((BN_ROWS, 1), lambda p, i: (i, 0)),
                  pl.BlockSpec((C, 2 * C), lambda p, i: (0, 0)),
                  pl.BlockSpec((C, 2 * C), lambda p, i: (0, 0))],
        out_specs=(pl.BlockSpec((BN_ROWS, C), lambda p, i: (i, 0)),
                   pl.BlockSpec((G, C), lambda p, i: (0, 0)),
                   pl.BlockSpec((BN_ROWS, 2 * C), lambda p, i: (i, 0)),
                   pl.BlockSpec((BN_ROWS, 2 * C), lambda p, i: (i, 0))),
        out_shape=(jax.ShapeDtypeStruct((N, C), jnp.float32),
                   jax.ShapeDtypeStruct((G, C), jnp.float32),
                   jax.ShapeDtypeStruct((N, 2 * C), jnp.float32),
                   jax.ShapeDtypeStruct((N, 2 * C), jnp.float32)),
        scratch_shapes=[pltpu.VMEM((8, C), jnp.float32)],
    )(parts, parts, h, gamma, beta, batch_f, wd, wsrc)


def _bn_body(agg_ref, h_ref, st_ref, gam_ref, bet_ref, b_ref, hn_ref, gf_ref):
    st = st_ref[...]
    mean = st[0:1, :] * (1.0 / N)
    var = st[1:2, :] * (1.0 / N) - mean * mean
    a = agg_ref[...]
    o = (a - mean) / jnp.sqrt(var + 1e-5) * gam_ref[...] + bet_ref[...] + h_ref[...]
    hn = jnp.maximum(o, 0.0)
    hn_ref[...] = hn

    @pl.when(pl.program_id(0) == 0)
    def _():
        gf_ref[...] = jnp.full((G, C), -jnp.inf, jnp.float32)

    b = b_ref[...]
    ms = [jnp.max(jnp.where(b == g, hn, -jnp.inf), axis=0, keepdims=True)
          for g in range(G)]
    gf_ref[...] = jnp.maximum(gf_ref[...], jnp.concatenate(ms, axis=0))


def _bn_relu_segmax(agg, h, st, gamma, beta, batch_f):
    return pl.pallas_call(
        _bn_body,
        grid=(NB_N,),
        in_specs=[pl.BlockSpec((BN_ROWS, C), lambda i: (i, 0)),
                  pl.BlockSpec((BN_ROWS, C), lambda i: (i, 0)),
                  pl.BlockSpec((8, C), lambda i: (0, 0)),
                  pl.BlockSpec((1, C), lambda i: (0, 0)),
                  pl.BlockSpec((1, C), lambda i: (0, 0)),
                  pl.BlockSpec((BN_ROWS, 1), lambda i: (i, 0))],
        out_specs=(pl.BlockSpec((BN_ROWS, C), lambda i: (i, 0)),
                   pl.BlockSpec((G, C), lambda i: (0, 0))),
        out_shape=(jax.ShapeDtypeStruct((N, C), jnp.float32),
                   jax.ShapeDtypeStruct((G, C), jnp.float32)),
    )(agg, h, st, gamma, beta, batch_f)


def _head_body(g0_ref, g1_ref, g2_ref, w1_ref, b1_ref, gam_ref, bet_ref,
               w2_ref, b2_ref, o_ref):
    gf = g0_ref[...] + g1_ref[...] + g2_ref[...]
    g = gf @ w1_ref[...] + b1_ref[...]
    m = jnp.mean(g, axis=0, keepdims=True)
    v = jnp.mean((g - m) ** 2, axis=0, keepdims=True)
    gr = jnp.maximum((g - m) / jnp.sqrt(v + 1e-5) * gam_ref[...] + bet_ref[...], 0.0)
    r = jnp.sum(gr * w2_ref[...], axis=1, keepdims=True) + b2_ref[0:1, 0:1]
    o_ref[...] = jnp.broadcast_to(r, (G, C))


def _head(g0, g1, g2, w1, b1, gam, bet, w2row, b2):
    full = lambda shape: pl.BlockSpec(shape, lambda: (0, 0))
    return pl.pallas_call(
        _head_body,
        in_specs=[full((G, C)), full((G, C)), full((G, C)), full((C, C)),
                  full((1, C)), full((1, C)), full((1, C)), full((1, C)),
                  full((8, C))],
        out_specs=full((G, C)),
        out_shape=jax.ShapeDtypeStruct((G, C), jnp.float32),
    )(g0, g1, g2, w1, b1, gam, bet, w2row, b2)


_MESH = plsc.VectorSubcoreMesh(core_axis_name="c", subcore_axis_name="s")

def _make_gather(nch, off):
    ne = nch * K * NW
    epw = nch * K

    @functools.partial(
        pl.kernel,
        mesh=_MESH,
        out_type=(jax.ShapeDtypeStruct((ne, 2 * C), jnp.float32),
                  jax.ShapeDtypeStruct((ne, 2 * C), jnp.float32)),
        scratch_types=[
            pltpu.VMEM((epw,), jnp.int32),
            pltpu.VMEM((epw,), jnp.int32),
            pltpu.VMEM((K, 2 * C), jnp.float32),
            pltpu.VMEM((K, 2 * C), jnp.float32),
            pltpu.VMEM((K, 2 * C), jnp.float32),
            pltpu.VMEM((K, 2 * C), jnp.float32),
            pltpu.SemaphoreType.DMA,
            pltpu.SemaphoreType.DMA,
        ],
    )
    def gather_k(td, ts, dst, src, gd, gs, idx_d, idx_s, rd0, rs0, rd1, rs1,
                 g0, g1):
        cid = lax.axis_index("c")
        sid = lax.axis_index("s")
        wid = sid * 2 + cid
        wbase = wid * epw
        pltpu.sync_copy(dst.at[pl.ds(off + wbase, epw)], idx_d)
        pltpu.sync_copy(src.at[pl.ds(off + wbase, epw)], idx_s)

        def issue(c, rd, rs, sem):
            pltpu.async_copy(td.at[idx_d.at[pl.ds(c * K, K)]], rd, sem)
            pltpu.async_copy(ts.at[idx_s.at[pl.ds(c * K, K)]], rs, sem)

        def finish(c, rd, rs, sem):
            pltpu.make_async_copy(td.at[idx_d.at[pl.ds(0, K)]], rd, sem).wait()
            pltpu.make_async_copy(ts.at[idx_s.at[pl.ds(0, K)]], rs, sem).wait()
            pltpu.sync_copy(rd, gd.at[pl.ds(wbase + c * K, K)])
            pltpu.sync_copy(rs, gs.at[pl.ds(wbase + c * K, K)])

        issue(0, rd0, rs0, g0)
        issue(1, rd1, rs1, g1)

        def body(j, carry):
            c0 = 2 * j
            finish(c0, rd0, rs0, g0)

            @pl.when(c0 + 2 < nch)
            def _():
                issue(c0 + 2, rd0, rs0, g0)

            @pl.when(c0 + 1 < nch)
            def _():
                finish(c0 + 1, rd1, rs1, g1)

                @pl.when(c0 + 3 < nch)
                def _():
                    issue(c0 + 3, rd1, rs1, g1)

            return carry

        lax.fori_loop(0, (nch + 1) // 2, body, 0)

    return gather_k


def _make_scatter(nch):
    ne = nch * K * NW
    epw = nch * K

    @functools.partial(
        pl.kernel,
        mesh=_MESH,
        out_type=jax.ShapeDtypeStruct((2 * N, C), jnp.float32),
        scratch_types=[
            pltpu.VMEM((nch, K), jnp.int32),
            pltpu.VMEM((K, C), jnp.float32),
            pltpu.VMEM((K, C), jnp.float32),
            pltpu.VMEM_SHARED((N, C), jnp.float32),
            pltpu.SemaphoreType.DMA,
            pltpu.SemaphoreType.DMA,
        ],
    )
    def scatter_k(msg, dst_r, zeros, out, idx_v, m0, m1, acc, s0, s1):
        cid = lax.axis_index("c")
        sid = lax.axis_index("s")
        pltpu.sync_copy(dst_r.at[cid, sid], idx_v)
        pltpu.sync_copy(zeros.at[pl.ds(sid * RPT, RPT)],
                        acc.at[pl.ds(sid * RPT, RPT)])

        @pl.when(sid == 0)
        def _():
            pltpu.sync_copy(zeros.at[pl.ds(16 * RPT, RTAIL)],
                            acc.at[pl.ds(16 * RPT, RTAIL)])

        plsc.subcore_barrier()
        ebase = cid * (ne // 2) + sid * epw

        def load(c, m, sem):
            pltpu.async_copy(msg.at[pl.ds(ebase + c * K, K)], m, sem)

        def flush(c, m, sem):
            pltpu.make_async_copy(msg.at[pl.ds(ebase, K)], m, sem).wait()
            pltpu.sync_copy(m, acc.at[idx_v.at[c]], add=True)

        load(0, m0, s0)
        load(1, m1, s1)

        def body(j, carry):
            c0 = 2 * j
            flush(c0, m0, s0)

            @pl.when(c0 + 2 < nch)
            def _():
                load(c0 + 2, m0, s0)

            @pl.when(c0 + 1 < nch)
            def _():
                flush(c0 + 1, m1, s1)

                @pl.when(c0 + 3 < nch)
                def _():
                    load(c0 + 3, m1, s1)

            return carry

        lax.fori_loop(0, (nch + 1) // 2, body, 0)
        plsc.subcore_barrier()
        pltpu.sync_copy(acc.at[pl.ds(sid * RPT, RPT)],
                        out.at[pl.ds(cid * N + sid * RPT, RPT)])

        @pl.when(sid == 0)
        def _():
            pltpu.sync_copy(acc.at[pl.ds(16 * RPT, RTAIL)],
                            out.at[pl.ds(cid * N + 16 * RPT, RTAIL)])

    return scatter_k


_GATHER_F = _make_gather(NCH, 0)
_SCATTER_F = _make_scatter(NCH)


def kernel(x, edge_index, edge_attr, batch, params):
    src = edge_index[0].astype(jnp.int32)
    dst = edge_index[1].astype(jnp.int32)
    dst_r = dst.reshape(2, 16, NCH, K)
    batch_f = batch.astype(jnp.float32).reshape(N, 1)
    zeros = jnp.zeros((N, C), jnp.float32)

    wds = []
    wsrcs = []
    for l in range(L):
        wf = params[f'conv{l}_Wf']
        ws = params[f'conv{l}_Ws']
        wds.append(jnp.concatenate([wf[:C], ws[:C]], axis=1))
        wsrcs.append(jnp.concatenate([wf[C:2 * C], ws[C:2 * C]], axis=1))

    h, td, ts = _embed(x, params['emb_W0'], params['emb_W1'], wds[0], wsrcs[0])
    gfs = []
    for l in range(L):
        wf = params[f'conv{l}_Wf']
        ws = params[f'conv{l}_Ws']
        bf = params[f'conv{l}_bf'].reshape(1, C)
        bs = params[f'conv{l}_bs'].reshape(1, C)
        gamma = params[f'conv{l}_gamma'].reshape(1, C)
        beta = params[f'conv{l}_beta'].reshape(1, C)
        gd, gs = _GATHER_F(td, ts, dst, src)
        msg = _edge_msg(gd, gs, edge_attr, wf[2 * C:], ws[2 * C:], bf, bs)
        parts = _SCATTER_F(msg, dst_r, zeros)
        if l < L - 1:
            h, gf, td, ts = _bn_nt(parts, h, gamma, beta, batch_f,
                                   wds[l + 1], wsrcs[l + 1])
        else:
            agg, st = _agg_stats(parts)
            h, gf = _bn_relu_segmax(agg, h, st, gamma, beta, batch_f)
        gfs.append(gf)

    out = _head(gfs[0], gfs[1], gfs[2], params['lin1_W'],
                params['lin1_b'].reshape(1, C),
                params['bn_gamma'].reshape(1, C),
                params['bn_beta'].reshape(1, C),
                params['lin2_W'].reshape(1, C),
                jnp.broadcast_to(params['lin2_b'].reshape(1, 1), (8, C)))
    return out[:, 0]
